# baseline (speedup 1.0000x reference)
import functools

import jax
import jax.numpy as jnp
from jax import lax
from jax.experimental import pallas as pl
from jax.experimental.pallas import tpu as pltpu
from jax.experimental.pallas import tpu_sc as plsc

R = 1024
I = 512
J = 256
K = 128
L = 16
W = 64
G = W // L
IH = 256
RING = 3
NW = 32
UNITS = J * (K // W) // NW
KW = K // W


def _scatter_body(in_hbm, idx_hbm, val_hbm, out_hbm,
                  buf, ix0, ix1, ix2, valbuf,
                  in_s, out_s, ix_s0, ix_s1, ix_s2):
    wid = lax.axis_index("s") * 2 + lax.axis_index("c")

    pltpu.sync_copy(val_hbm, valbuf)
    val = valbuf[...]
    cols = [lax.iota(jnp.int32, L) + g * L for g in range(G)]

    rings = [ix0, ix1, ix2]
    ix_s = [ix_s0, ix_s1, ix_s2]

    def unit_jk(u):
        unit = wid * UNITS + u
        return unit // KW, (unit % KW) * W

    def start_ix(tok):
        u, h = tok // 2, tok % 2
        j, k0 = unit_jk(u)
        slot = tok % RING
        return pltpu.async_copy(
            idx_hbm.at[pl.ds(h * IH, IH), j, pl.ds(k0, W)],
            rings[slot], ix_s[slot])

    def start_in(u):
        j, k0 = unit_jk(u)
        return pltpu.async_copy(in_hbm.at[:, j, pl.ds(k0, W)], buf, in_s)

    ix_h = {0: start_ix(0), 1: start_ix(1)}
    in_h = start_in(0)
    for u in range(UNITS):
        in_h.wait()
        for h in range(2):
            tok = 2 * u + h
            slot = tok % RING
            ix_h.pop(tok).wait()
            if tok + 2 < 2 * UNITS:
                ix_h[tok + 2] = start_ix(tok + 2)

            def body(i, carry):
                for g in range(G):
                    rows = plsc.bitcast(
                        rings[slot][i, pl.ds(g * L, L)], jnp.int32)
                    plsc.store_scatter(buf, [rows, cols[g]], val)
                return carry

            lax.fori_loop(0, IH, body, 0)
        j, k0 = unit_jk(u)
        out_h = pltpu.async_copy(buf, out_hbm.at[:, j, pl.ds(k0, W)], out_s)
        out_h.wait()
        if u + 1 < UNITS:
            in_h = start_in(u + 1)


def kernel(input, index, value):
    idx = index.astype(jnp.uint32)
    val = jnp.full((L,), value, jnp.float32)

    mesh = plsc.VectorSubcoreMesh(core_axis_name="c", subcore_axis_name="s")
    run = functools.partial(
        pl.kernel,
        mesh=mesh,
        out_type=jax.ShapeDtypeStruct((R, J, K), jnp.float32),
        scratch_types=[
            pltpu.VMEM((R, W), jnp.float32),
            pltpu.VMEM((IH, W), jnp.uint32),
            pltpu.VMEM((IH, W), jnp.uint32),
            pltpu.VMEM((IH, W), jnp.uint32),
            pltpu.VMEM((L,), jnp.float32),
            pltpu.SemaphoreType.DMA,
            pltpu.SemaphoreType.DMA,
            pltpu.SemaphoreType.DMA,
            pltpu.SemaphoreType.DMA,
            pltpu.SemaphoreType.DMA,
        ],
        compiler_params=pltpu.CompilerParams(
            use_tc_tiling_on_sc=False,
            needs_layout_passes=False,
        ),
    )(_scatter_body)
    return run(input, idx, val)


# speedup vs baseline: 1.0683x; 1.0683x over previous
import functools

import jax
import jax.numpy as jnp
from jax import lax
from jax.experimental import pallas as pl
from jax.experimental.pallas import tpu as pltpu
from jax.experimental.pallas import tpu_sc as plsc

R = 1024
I = 512
J = 256
K = 128
L = 16
W = 32
G = W // L
NW = 32
UNITS = J * (K // W) // NW
KW = K // W


def _scatter_body(in_hbm, idx_hbm, val_hbm, out_hbm,
                  buf0, buf1, idx0, idx1, valbuf,
                  in_s0, in_s1, ix_s0, ix_s1, out_s0, out_s1):
    wid = lax.axis_index("s") * 2 + lax.axis_index("c")

    pltpu.sync_copy(val_hbm, valbuf)
    val = valbuf[...]
    cols = [lax.iota(jnp.int32, L) + g * L for g in range(G)]

    bufs = [buf0, buf1]
    idxs = [idx0, idx1]
    in_s = [in_s0, in_s1]
    ix_s = [ix_s0, ix_s1]
    out_s = [out_s0, out_s1]

    def unit_jk(u):
        unit = wid * UNITS + u
        return unit // KW, (unit % KW) * W

    def start_loads(u, s):
        j, k0 = unit_jk(u)
        hi = pltpu.async_copy(in_hbm.at[:, j, pl.ds(k0, W)], bufs[s], in_s[s])
        hx = pltpu.async_copy(idx_hbm.at[:, j, pl.ds(k0, W)], idxs[s], ix_s[s])
        return hi, hx

    loads = {0: start_loads(0, 0)}
    stores = {}
    for u in range(UNITS):
        s = u % 2
        hi, hx = loads.pop(u)
        hi.wait()
        hx.wait()
        if u + 1 < UNITS:
            s2 = (u + 1) % 2
            if u >= 1:
                stores.pop(u - 1).wait()
            loads[u + 1] = start_loads(u + 1, s2)

        def body(i, carry):
            i2 = i * 2
            for r in range(2):
                for g in range(G):
                    rows = plsc.bitcast(
                        idxs[s][i2 + r, pl.ds(g * L, L)], jnp.int32)
                    plsc.store_scatter(bufs[s], [rows, cols[g]], val)
            return carry

        lax.fori_loop(jnp.int32(0), jnp.int32(I // 2), body, 0)
        j, k0 = unit_jk(u)
        stores[u] = pltpu.async_copy(bufs[s], out_hbm.at[:, j, pl.ds(k0, W)],
                                     out_s[s])
    stores.pop(UNITS - 2).wait()
    stores.pop(UNITS - 1).wait()


def kernel(input, index, value):
    idx = index.astype(jnp.uint32)
    val = jnp.full((L,), value, jnp.float32)

    mesh = plsc.VectorSubcoreMesh(core_axis_name="c", subcore_axis_name="s")
    run = functools.partial(
        pl.kernel,
        mesh=mesh,
        out_type=jax.ShapeDtypeStruct((R, J, K), jnp.float32),
        scratch_types=[
            pltpu.VMEM((R, W), jnp.float32),
            pltpu.VMEM((R, W), jnp.float32),
            pltpu.VMEM((I, W), jnp.uint32),
            pltpu.VMEM((I, W), jnp.uint32),
            pltpu.VMEM((L,), jnp.float32),
            pltpu.SemaphoreType.DMA,
            pltpu.SemaphoreType.DMA,
            pltpu.SemaphoreType.DMA,
            pltpu.SemaphoreType.DMA,
            pltpu.SemaphoreType.DMA,
            pltpu.SemaphoreType.DMA,
        ],
        compiler_params=pltpu.CompilerParams(
            use_tc_tiling_on_sc=False,
            needs_layout_passes=False,
        ),
    )(_scatter_body)
    return run(input, idx, val)


# speedup vs baseline: 1.0761x; 1.0073x over previous
import functools

import jax
import jax.numpy as jnp
from jax import lax
from jax.experimental import pallas as pl
from jax.experimental.pallas import tpu as pltpu
from jax.experimental.pallas import tpu_sc as plsc

R = 1024
I = 512
J = 256
K = 128
L = 16
W = 32
G = W // L
NW = 32
UNITS = J * (K // W) // NW
KW = K // W


def _scatter_body(in_hbm, idx_hbm, val_hbm, out_hbm,
                  buf0, buf1, idx0, idx1, valbuf,
                  in_s0, in_s1, ix_s0, ix_s1, out_s0, out_s1):
    wid = lax.axis_index("s") * 2 + lax.axis_index("c")

    pltpu.sync_copy(val_hbm, valbuf)
    val = valbuf[...]
    cols = [lax.iota(jnp.int32, L) + g * L for g in range(G)]

    bufs = [buf0, buf1]
    idxs = [idx0, idx1]
    in_s = [in_s0, in_s1]
    ix_s = [ix_s0, ix_s1]
    out_s = [out_s0, out_s1]

    def unit_jk(u):
        unit = wid * UNITS + u
        return unit // KW, (unit % KW) * W

    def start_loads(u, s):
        j, k0 = unit_jk(u)
        hi = pltpu.async_copy(in_hbm.at[:, j, pl.ds(k0, W)], bufs[s], in_s[s])
        hx = pltpu.async_copy(idx_hbm.at[:, j, pl.ds(k0, W)], idxs[s], ix_s[s])
        return hi, hx

    loads = {0: start_loads(0, 0)}
    stores = {}
    for u in range(UNITS):
        s = u % 2
        hi, hx = loads.pop(u)
        hi.wait()
        hx.wait()
        if u + 1 < UNITS:
            s2 = (u + 1) % 2
            if u >= 1:
                stores.pop(u - 1).wait()
            loads[u + 1] = start_loads(u + 1, s2)

        def body(i, carry):
            i2 = i * 4
            for r in range(4):
                for g in range(G):
                    rows = plsc.bitcast(
                        idxs[s][i2 + r, pl.ds(g * L, L)], jnp.int32)
                    plsc.store_scatter(bufs[s], [rows, cols[g]], val)
            return carry

        lax.fori_loop(jnp.int32(0), jnp.int32(I // 4), body, 0)
        j, k0 = unit_jk(u)
        stores[u] = pltpu.async_copy(bufs[s], out_hbm.at[:, j, pl.ds(k0, W)],
                                     out_s[s])
    stores.pop(UNITS - 2).wait()
    stores.pop(UNITS - 1).wait()


def kernel(input, index, value):
    idx = index.astype(jnp.uint32)
    val = jnp.full((L,), value, jnp.float32)

    mesh = plsc.VectorSubcoreMesh(core_axis_name="c", subcore_axis_name="s")
    run = functools.partial(
        pl.kernel,
        mesh=mesh,
        out_type=jax.ShapeDtypeStruct((R, J, K), jnp.float32),
        scratch_types=[
            pltpu.VMEM((R, W), jnp.float32),
            pltpu.VMEM((R, W), jnp.float32),
            pltpu.VMEM((I, W), jnp.uint32),
            pltpu.VMEM((I, W), jnp.uint32),
            pltpu.VMEM((L,), jnp.float32),
            pltpu.SemaphoreType.DMA,
            pltpu.SemaphoreType.DMA,
            pltpu.SemaphoreType.DMA,
            pltpu.SemaphoreType.DMA,
            pltpu.SemaphoreType.DMA,
            pltpu.SemaphoreType.DMA,
        ],
        compiler_params=pltpu.CompilerParams(
            use_tc_tiling_on_sc=False,
            needs_layout_passes=False,
        ),
    )(_scatter_body)
    return run(input, idx, val)
